# baseline (speedup 1.0000x reference)
import functools

import jax
import jax.numpy as jnp
from jax import lax
from jax.experimental import pallas as pl
from jax.experimental.pallas import tpu as pltpu
from jax.experimental.pallas import tpu_sc as plsc

NC = 2
NS = 16
NW = NC * NS
LANES = 16


def _make_prep_sc_kernel(Npad, E, D):
    EPT = E // NS
    SL = Npad // NS
    YR = Npad // NW
    mesh = plsc.VectorSubcoreMesh(
        core_axis_name="c", subcore_axis_name="s", num_cores=NC, num_subcores=NS)

    @functools.partial(
        pl.kernel,
        out_type=(
            jax.ShapeDtypeStruct((Npad, D), jnp.float32),
            jax.ShapeDtypeStruct((Npad,), jnp.float32),
        ),
        mesh=mesh,
        scratch_types=[
            pltpu.VMEM((Npad,), jnp.float32),
            pltpu.VMEM((EPT,), jnp.int32),
            pltpu.VMEM((SL,), jnp.float32),
            pltpu.VMEM((SL,), jnp.float32),
            pltpu.VMEM((SL,), jnp.float32),
            pltpu.VMEM((YR,), jnp.float32),
            pltpu.VMEM((YR, D), jnp.float32),
            pltpu.VMEM_SHARED((NS, Npad), jnp.float32),
            pltpu.VMEM_SHARED((Npad,), jnp.float32),
        ],
        compiler_params=pltpu.CompilerParams(
            needs_layout_passes=False, use_tc_tiling_on_sc=False),
    )
    def prep_sc(src_hbm, x_hbm, y_hbm, r_hbm,
                deg_v, src_v, acc_v, part_v, r_v, rloc_v, xrow_v,
                slab_sh, r_sh):
        c = lax.axis_index("c")
        s = lax.axis_index("s")
        wid = s * NC + c

        zeros = jnp.zeros((LANES,), jnp.float32)

        def zero_body(i, carry):
            deg_v[pl.ds(i * LANES, LANES)] = zeros
            return carry

        lax.fori_loop(0, Npad // LANES, zero_body, None)

        pltpu.sync_copy(src_hbm.at[pl.ds(s * EPT, EPT)], src_v)

        ones = jnp.ones((LANES,), jnp.float32)

        def count_body(i, carry):
            idx = src_v[pl.ds(i * LANES, LANES)]
            plsc.addupdate_scatter(deg_v, [idx], ones)
            return carry

        lax.fori_loop(0, EPT // LANES, count_body, None)

        pltpu.sync_copy(deg_v, slab_sh.at[s])
        plsc.subcore_barrier()

        pltpu.sync_copy(slab_sh.at[0, pl.ds(s * SL, SL)], acc_v)
        for k in range(1, NS):
            pltpu.sync_copy(slab_sh.at[k, pl.ds(s * SL, SL)], part_v)

            def add_body(g, carry):
                sl = pl.ds(g * LANES, LANES)
                acc_v[sl] = acc_v[sl] + part_v[sl]
                return carry

            lax.fori_loop(0, SL // LANES, add_body, None)

        def rsqrt_body(g, carry):
            sl = pl.ds(g * LANES, LANES)
            d = acc_v[sl]
            i = plsc.bitcast(d, jnp.int32)
            i = jnp.int32(0x5F3759DF) - lax.shift_right_arithmetic(
                i, jnp.int32(1))
            yv = plsc.bitcast(i, jnp.float32)
            hd = 0.5 * d
            yv = yv * (1.5 - hd * yv * yv)
            yv = yv * (1.5 - hd * yv * yv)
            yv = yv * (1.5 - hd * yv * yv)
            r_v[sl] = yv
            return carry

        lax.fori_loop(0, SL // LANES, rsqrt_body, None)

        pltpu.sync_copy(r_v, r_sh.at[pl.ds(s * SL, SL)])

        @pl.when(c == 0)
        def _():
            pltpu.sync_copy(r_v, r_hbm.at[pl.ds(s * SL, SL)])

        plsc.subcore_barrier()

        base = wid * YR
        pltpu.sync_copy(r_sh.at[pl.ds(base, YR)], rloc_v)
        pltpu.sync_copy(x_hbm.at[pl.ds(base, YR)], xrow_v)

        def scale_row(i, carry):
            iv = jnp.full((LANES,), i, jnp.int32)
            br = plsc.load_gather(rloc_v, [iv])
            for jj in range(D // LANES):
                sl = pl.ds(jj * LANES, LANES)
                xrow_v[i, sl] = xrow_v[i, sl] * br
            return carry

        lax.fori_loop(0, YR, scale_row, None)

        pltpu.sync_copy(xrow_v, y_hbm.at[pl.ds(base, YR)])

    return prep_sc


def _make_agg_kernel(N, E, D, CH):
    EPW = E // NW
    NCHK = EPW // CH
    R = N // NS
    ZR = 25
    mesh = plsc.VectorSubcoreMesh(
        core_axis_name="c", subcore_axis_name="s", num_cores=NC, num_subcores=NS)

    @functools.partial(
        pl.kernel,
        out_type=jax.ShapeDtypeStruct((NC, N, D), jnp.float32),
        mesh=mesh,
        scratch_types=[
            pltpu.VMEM((NCHK, CH), jnp.int32),
            pltpu.VMEM((NCHK, CH), jnp.int32),
            pltpu.VMEM((CH, D), jnp.float32),
            pltpu.VMEM((CH, D), jnp.float32),
            pltpu.VMEM((25, D), jnp.float32),
            pltpu.VMEM_SHARED((N, D), jnp.float32),
            pltpu.SemaphoreType.DMA,
            pltpu.SemaphoreType.DMA,
            pltpu.SemaphoreType.DMA,
            pltpu.SemaphoreType.DMA,
        ],
        compiler_params=pltpu.CompilerParams(
            needs_layout_passes=False, use_tc_tiling_on_sc=False),
    )
    def agg_kernel(y_hbm, src_hbm, dst_hbm, out_hbm,
                   src_v, dst_v, rows0_v, rows1_v, zbuf_v, acc_sh,
                   s0a, s0b, s1a, s1b):
        c = lax.axis_index("c")
        s = lax.axis_index("s")
        wid = s * NC + c

        zeros = jnp.zeros((LANES,), jnp.float32)

        def zrow(i, carry):
            def zcol(j, cc):
                zbuf_v[i, pl.ds(j * LANES, LANES)] = zeros
                return cc
            return lax.fori_loop(0, D // LANES, zcol, carry)

        lax.fori_loop(0, ZR, zrow, None)
        for k in range(R // ZR):
            pltpu.sync_copy(zbuf_v, acc_sh.at[pl.ds(s * R + k * ZR, ZR)])

        pltpu.sync_copy(src_hbm.at[pl.ds(wid * NCHK, NCHK)], src_v)
        pltpu.sync_copy(dst_hbm.at[pl.ds(wid * NCHK, NCHK)], dst_v)

        plsc.subcore_barrier()

        H = CH // 2

        def gather(j, buf, sa, sb):
            pltpu.async_copy(
                y_hbm.at[src_v.at[j, pl.ds(0, H)]], buf.at[pl.ds(0, H)], sa)
            pltpu.async_copy(
                y_hbm.at[src_v.at[j, pl.ds(H, H)]], buf.at[pl.ds(H, H)], sb)

        def drain(buf, sa, sb):
            pltpu.make_async_copy(
                y_hbm.at[pl.ds(0, H)], buf.at[pl.ds(0, H)], sa).wait()
            pltpu.make_async_copy(
                y_hbm.at[pl.ds(0, H)], buf.at[pl.ds(H, H)], sb).wait()

        def scat(j, buf):
            pltpu.sync_copy(buf, acc_sh.at[dst_v.at[j]], add=True)

        gather(0, rows0_v, s0a, s0b)

        def pair_body(g, carry):
            gather(2 * g + 1, rows1_v, s1a, s1b)
            drain(rows0_v, s0a, s0b)
            scat(2 * g, rows0_v)
            gather(jnp.minimum(2 * g + 2, NCHK - 1), rows0_v, s0a, s0b)
            drain(rows1_v, s1a, s1b)
            scat(2 * g + 1, rows1_v)
            return carry

        lax.fori_loop(0, NCHK // 2, pair_body, None)
        drain(rows0_v, s0a, s0b)
        if NCHK % 2 == 1:
            scat(NCHK - 1, rows0_v)

        plsc.subcore_barrier()

        pltpu.sync_copy(acc_sh.at[pl.ds(s * R, R)], out_hbm.at[c, pl.ds(s * R, R)])

    return agg_kernel


def _make_final_kernel(N, D, BLK):
    def final_body(x_ref, acc_ref, r_ref, wl_ref, wh_ref, wm_ref,
                   al_ref, ah_ref, am_ref, att_ref, out_ref):
        r = r_ref[...]
        low = (acc_ref[0] + acc_ref[1]) * r
        x = x_ref[...]
        high = x - low
        lowW = jnp.maximum(
            jnp.dot(low, wl_ref[...], preferred_element_type=jnp.float32), 0.0)
        highW = jnp.maximum(
            jnp.dot(high, wh_ref[...], preferred_element_type=jnp.float32), 0.0)
        mlpW = jnp.maximum(
            jnp.dot(x, wm_ref[...], preferred_element_type=jnp.float32), 0.0)

        l0 = jnp.dot(lowW, al_ref[...], preferred_element_type=jnp.float32)
        l1 = jnp.dot(highW, ah_ref[...], preferred_element_type=jnp.float32)
        l2 = jnp.dot(mlpW, am_ref[...], preferred_element_type=jnp.float32)

        s0 = 1.0 / (1.0 + jnp.exp(-l0))
        s1 = 1.0 / (1.0 + jnp.exp(-l1))
        s2 = 1.0 / (1.0 + jnp.exp(-l2))

        T_inv = 1.0 / 3.0
        m0 = (s0 * att_ref[0, 0] + s1 * att_ref[1, 0] + s2 * att_ref[2, 0]) * T_inv
        m1 = (s0 * att_ref[0, 1] + s1 * att_ref[1, 1] + s2 * att_ref[2, 1]) * T_inv
        m2 = (s0 * att_ref[0, 2] + s1 * att_ref[1, 2] + s2 * att_ref[2, 2]) * T_inv

        mx = jnp.maximum(jnp.maximum(m0, m1), m2)
        e0 = jnp.exp(m0 - mx)
        e1 = jnp.exp(m1 - mx)
        e2 = jnp.exp(m2 - mx)
        den = e0 + e1 + e2

        out_ref[...] = 3.0 * ((e0 / den) * lowW + (e1 / den) * highW +
                              (e2 / den) * mlpW)

    return pl.pallas_call(
        final_body,
        grid=(N // BLK,),
        in_specs=[
            pl.BlockSpec((BLK, D), lambda i: (i, 0)),
            pl.BlockSpec((NC, BLK, D), lambda i: (0, i, 0)),
            pl.BlockSpec((BLK, 1), lambda i: (i, 0)),
            pl.BlockSpec((D, D), lambda i: (0, 0)),
            pl.BlockSpec((D, D), lambda i: (0, 0)),
            pl.BlockSpec((D, D), lambda i: (0, 0)),
            pl.BlockSpec((D, 1), lambda i: (0, 0)),
            pl.BlockSpec((D, 1), lambda i: (0, 0)),
            pl.BlockSpec((D, 1), lambda i: (0, 0)),
            pl.BlockSpec(memory_space=pltpu.SMEM),
        ],
        out_specs=pl.BlockSpec((BLK, D), lambda i: (i, 0)),
        out_shape=jax.ShapeDtypeStruct((N, D), jnp.float32),
    )


def kernel(x, edge_index, W_low, W_high, W_mlp, a_low, a_high, a_mlp, att_vec):
    N, D = x.shape
    E = edge_index.shape[1]
    CH = 80
    BLK = 2000

    src = edge_index[0]
    dst = edge_index[1]
    src_ch = src.reshape(E // CH, CH)
    dst_ch = dst.reshape(E // CH, CH)

    Npad = 10240
    x_pad = jnp.pad(x, ((0, Npad - N), (0, 0)))

    y_pad, r_pad = _make_prep_sc_kernel(Npad, E, D)(src, x_pad)
    r = r_pad[:N].reshape(N, 1)
    acc = _make_agg_kernel(N, E, D, CH)(y_pad, src_ch, dst_ch)

    return _make_final_kernel(N, D, BLK)(
        x, acc, r, W_low, W_high, W_mlp, a_low, a_high, a_mlp, att_vec)


# speedup vs baseline: 1.0333x; 1.0333x over previous
import functools

import jax
import jax.numpy as jnp
from jax import lax
from jax.experimental import pallas as pl
from jax.experimental.pallas import tpu as pltpu
from jax.experimental.pallas import tpu_sc as plsc

NC = 2
NS = 16
NW = NC * NS
LANES = 16


def _make_deg_kernel(N, E):
    EPW = E // NW
    mesh = plsc.VectorSubcoreMesh(
        core_axis_name="c", subcore_axis_name="s", num_cores=NC, num_subcores=NS)

    @functools.partial(
        pl.kernel,
        out_type=jax.ShapeDtypeStruct((NW * N,), jnp.float32),
        mesh=mesh,
        scratch_types=[
            pltpu.VMEM((N,), jnp.float32),
            pltpu.VMEM((EPW,), jnp.int32),
        ],
        compiler_params=pltpu.CompilerParams(
            needs_layout_passes=False, use_tc_tiling_on_sc=False),
    )
    def deg_kernel(src_hbm, out_hbm, deg_v, src_v):
        c = lax.axis_index("c")
        s = lax.axis_index("s")
        wid = s * NC + c

        zeros = jnp.zeros((LANES,), jnp.float32)

        def zero_body(i, carry):
            deg_v[pl.ds(i * LANES, LANES)] = zeros
            return carry

        lax.fori_loop(0, N // LANES, zero_body, None)

        pltpu.sync_copy(src_hbm.at[pl.ds(wid * EPW, EPW)], src_v)

        ones = jnp.ones((LANES,), jnp.float32)

        def count_body(i, carry):
            idx = src_v[pl.ds(i * LANES, LANES)]
            plsc.addupdate_scatter(deg_v, [idx], ones)
            return carry

        lax.fori_loop(0, EPW // LANES, count_body, None)

        pltpu.sync_copy(deg_v, out_hbm.at[pl.ds(wid * N, N)])

    return deg_kernel


def _make_prep_kernel(N, D, BLK):
    def prep_body(x_ref, degp_ref, y_ref, r_ref):
        deg = jnp.sum(degp_ref[...], axis=1, keepdims=True)
        r = lax.rsqrt(deg)
        r_ref[...] = r
        y_ref[...] = x_ref[...] * r

    return pl.pallas_call(
        prep_body,
        grid=(N // BLK,),
        in_specs=[
            pl.BlockSpec((BLK, D), lambda i: (i, 0)),
            pl.BlockSpec((BLK, NW), lambda i: (i, 0)),
        ],
        out_specs=[
            pl.BlockSpec((BLK, D), lambda i: (i, 0)),
            pl.BlockSpec((BLK, 1), lambda i: (i, 0)),
        ],
        out_shape=[
            jax.ShapeDtypeStruct((N, D), jnp.float32),
            jax.ShapeDtypeStruct((N, 1), jnp.float32),
        ],
    )


def _make_agg_kernel(N, E, D, CH):
    EPW = E // NW
    NCHK = EPW // CH
    R = N // NS
    ZR = 25
    mesh = plsc.VectorSubcoreMesh(
        core_axis_name="c", subcore_axis_name="s", num_cores=NC, num_subcores=NS)

    @functools.partial(
        pl.kernel,
        out_type=jax.ShapeDtypeStruct((NC, N, D), jnp.float32),
        mesh=mesh,
        scratch_types=[
            pltpu.VMEM((NCHK, CH), jnp.int32),
            pltpu.VMEM((NCHK, CH), jnp.int32),
            pltpu.VMEM((CH, D), jnp.float32),
            pltpu.VMEM((CH, D), jnp.float32),
            pltpu.VMEM((25, D), jnp.float32),
            pltpu.VMEM_SHARED((N, D), jnp.float32),
            pltpu.SemaphoreType.DMA,
            pltpu.SemaphoreType.DMA,
            pltpu.SemaphoreType.DMA,
            pltpu.SemaphoreType.DMA,
        ],
        compiler_params=pltpu.CompilerParams(
            needs_layout_passes=False, use_tc_tiling_on_sc=False),
    )
    def agg_kernel(y_hbm, src_hbm, dst_hbm, out_hbm,
                   src_v, dst_v, rows0_v, rows1_v, zbuf_v, acc_sh,
                   s0a, s0b, s1a, s1b):
        c = lax.axis_index("c")
        s = lax.axis_index("s")
        wid = s * NC + c

        zeros = jnp.zeros((LANES,), jnp.float32)

        def zrow(i, carry):
            def zcol(j, cc):
                zbuf_v[i, pl.ds(j * LANES, LANES)] = zeros
                return cc
            return lax.fori_loop(0, D // LANES, zcol, carry)

        lax.fori_loop(0, ZR, zrow, None)
        for k in range(R // ZR):
            pltpu.sync_copy(zbuf_v, acc_sh.at[pl.ds(s * R + k * ZR, ZR)])

        pltpu.sync_copy(src_hbm.at[pl.ds(wid * NCHK, NCHK)], src_v)
        pltpu.sync_copy(dst_hbm.at[pl.ds(wid * NCHK, NCHK)], dst_v)

        plsc.subcore_barrier()

        H = CH // 2

        def gather(j, buf, sa, sb):
            pltpu.async_copy(
                y_hbm.at[src_v.at[j, pl.ds(0, H)]], buf.at[pl.ds(0, H)], sa)
            pltpu.async_copy(
                y_hbm.at[src_v.at[j, pl.ds(H, H)]], buf.at[pl.ds(H, H)], sb)

        def drain(buf, sa, sb):
            pltpu.make_async_copy(
                y_hbm.at[pl.ds(0, H)], buf.at[pl.ds(0, H)], sa).wait()
            pltpu.make_async_copy(
                y_hbm.at[pl.ds(0, H)], buf.at[pl.ds(H, H)], sb).wait()

        def scat(j, buf):
            pltpu.sync_copy(buf, acc_sh.at[dst_v.at[j]], add=True)

        gather(0, rows0_v, s0a, s0b)

        def pair_body(g, carry):
            gather(2 * g + 1, rows1_v, s1a, s1b)
            drain(rows0_v, s0a, s0b)
            scat(2 * g, rows0_v)
            gather(jnp.minimum(2 * g + 2, NCHK - 1), rows0_v, s0a, s0b)
            drain(rows1_v, s1a, s1b)
            scat(2 * g + 1, rows1_v)
            return carry

        lax.fori_loop(0, NCHK // 2, pair_body, None)
        drain(rows0_v, s0a, s0b)
        if NCHK % 2 == 1:
            scat(NCHK - 1, rows0_v)

        plsc.subcore_barrier()

        pltpu.sync_copy(acc_sh.at[pl.ds(s * R, R)], out_hbm.at[c, pl.ds(s * R, R)])

    return agg_kernel


def _make_mlp_kernel(N, D, BLK):
    def mlp_body(x_ref, wm_ref, mlp_ref):
        mlp_ref[...] = jnp.maximum(
            jnp.dot(x_ref[...], wm_ref[...],
                    preferred_element_type=jnp.float32), 0.0)

    return pl.pallas_call(
        mlp_body,
        grid=(N // BLK,),
        in_specs=[
            pl.BlockSpec((BLK, D), lambda i: (i, 0)),
            pl.BlockSpec((D, D), lambda i: (0, 0)),
        ],
        out_specs=pl.BlockSpec((BLK, D), lambda i: (i, 0)),
        out_shape=jax.ShapeDtypeStruct((N, D), jnp.float32),
    )


def _make_final_kernel(N, D, BLK):
    def final_body(x_ref, acc_ref, r_ref, mlp_ref, wl_ref, wh_ref,
                   al_ref, ah_ref, am_ref, att_ref, out_ref):
        r = r_ref[...]
        low = (acc_ref[0] + acc_ref[1]) * r
        x = x_ref[...]
        high = x - low
        lowW = jnp.maximum(
            jnp.dot(low, wl_ref[...], preferred_element_type=jnp.float32), 0.0)
        highW = jnp.maximum(
            jnp.dot(high, wh_ref[...], preferred_element_type=jnp.float32), 0.0)
        mlpW = mlp_ref[...]

        l0 = jnp.dot(lowW, al_ref[...], preferred_element_type=jnp.float32)
        l1 = jnp.dot(highW, ah_ref[...], preferred_element_type=jnp.float32)
        l2 = jnp.dot(mlpW, am_ref[...], preferred_element_type=jnp.float32)

        s0 = 1.0 / (1.0 + jnp.exp(-l0))
        s1 = 1.0 / (1.0 + jnp.exp(-l1))
        s2 = 1.0 / (1.0 + jnp.exp(-l2))

        T_inv = 1.0 / 3.0
        m0 = (s0 * att_ref[0, 0] + s1 * att_ref[1, 0] + s2 * att_ref[2, 0]) * T_inv
        m1 = (s0 * att_ref[0, 1] + s1 * att_ref[1, 1] + s2 * att_ref[2, 1]) * T_inv
        m2 = (s0 * att_ref[0, 2] + s1 * att_ref[1, 2] + s2 * att_ref[2, 2]) * T_inv

        mx = jnp.maximum(jnp.maximum(m0, m1), m2)
        e0 = jnp.exp(m0 - mx)
        e1 = jnp.exp(m1 - mx)
        e2 = jnp.exp(m2 - mx)
        den = e0 + e1 + e2

        out_ref[...] = 3.0 * ((e0 / den) * lowW + (e1 / den) * highW +
                              (e2 / den) * mlpW)

    return pl.pallas_call(
        final_body,
        grid=(N // BLK,),
        in_specs=[
            pl.BlockSpec((BLK, D), lambda i: (i, 0)),
            pl.BlockSpec((NC, BLK, D), lambda i: (0, i, 0)),
            pl.BlockSpec((BLK, 1), lambda i: (i, 0)),
            pl.BlockSpec((BLK, D), lambda i: (i, 0)),
            pl.BlockSpec((D, D), lambda i: (0, 0)),
            pl.BlockSpec((D, D), lambda i: (0, 0)),
            pl.BlockSpec((D, 1), lambda i: (0, 0)),
            pl.BlockSpec((D, 1), lambda i: (0, 0)),
            pl.BlockSpec((D, 1), lambda i: (0, 0)),
            pl.BlockSpec(memory_space=pltpu.SMEM),
        ],
        out_specs=pl.BlockSpec((BLK, D), lambda i: (i, 0)),
        out_shape=jax.ShapeDtypeStruct((N, D), jnp.float32),
    )


def kernel(x, edge_index, W_low, W_high, W_mlp, a_low, a_high, a_mlp, att_vec):
    N, D = x.shape
    E = edge_index.shape[1]
    CH = 80
    BLK = 2000

    src = edge_index[0]
    dst = edge_index[1]
    src_ch = src.reshape(E // CH, CH)
    dst_ch = dst.reshape(E // CH, CH)

    deg_p = _make_deg_kernel(N, E)(src).reshape(NW, N)
    deg_pt = deg_p.T

    y, r = _make_prep_kernel(N, D, BLK)(x, deg_pt)
    acc = _make_agg_kernel(N, E, D, CH)(y, src_ch, dst_ch)
    mlp = _make_mlp_kernel(N, D, BLK)(x, W_mlp)

    return _make_final_kernel(N, D, BLK)(
        x, acc, r, mlp, W_low, W_high, a_low, a_high, a_mlp, att_vec)


# speedup vs baseline: 1.0511x; 1.0172x over previous
import functools

import jax
import jax.numpy as jnp
from jax import lax
from jax.experimental import pallas as pl
from jax.experimental.pallas import tpu as pltpu
from jax.experimental.pallas import tpu_sc as plsc

NC = 2
NS = 16
NW = NC * NS
LANES = 16


def _make_deg_kernel(N, E):
    EPW = E // NW
    mesh = plsc.VectorSubcoreMesh(
        core_axis_name="c", subcore_axis_name="s", num_cores=NC, num_subcores=NS)

    @functools.partial(
        pl.kernel,
        out_type=jax.ShapeDtypeStruct((NW * N,), jnp.float32),
        mesh=mesh,
        scratch_types=[
            pltpu.VMEM((N,), jnp.float32),
            pltpu.VMEM((EPW,), jnp.int32),
        ],
        compiler_params=pltpu.CompilerParams(
            needs_layout_passes=False, use_tc_tiling_on_sc=False),
    )
    def deg_kernel(src_hbm, out_hbm, deg_v, src_v):
        c = lax.axis_index("c")
        s = lax.axis_index("s")
        wid = s * NC + c

        zeros = jnp.zeros((LANES,), jnp.float32)

        def zero_body(i, carry):
            deg_v[pl.ds(i * LANES, LANES)] = zeros
            return carry

        lax.fori_loop(0, N // LANES, zero_body, None)

        pltpu.sync_copy(src_hbm.at[pl.ds(wid * EPW, EPW)], src_v)

        ones = jnp.ones((LANES,), jnp.float32)

        def count_body(i, carry):
            idx = src_v[pl.ds(i * LANES, LANES)]
            plsc.addupdate_scatter(deg_v, [idx], ones)
            return carry

        lax.fori_loop(0, EPW // LANES, count_body, None)

        pltpu.sync_copy(deg_v, out_hbm.at[pl.ds(wid * N, N)])

    return deg_kernel


def _make_prep_kernel(N, D, BLK):
    def prep_body(x_ref, degp_ref, y_ref, r_ref):
        deg = jnp.sum(degp_ref[...], axis=1, keepdims=True)
        r = lax.rsqrt(deg)
        r_ref[...] = r
        y_ref[...] = x_ref[...] * r

    return pl.pallas_call(
        prep_body,
        grid=(N // BLK,),
        in_specs=[
            pl.BlockSpec((BLK, D), lambda i: (i, 0)),
            pl.BlockSpec((BLK, NW), lambda i: (i, 0)),
        ],
        out_specs=[
            pl.BlockSpec((BLK, D), lambda i: (i, 0)),
            pl.BlockSpec((BLK, 1), lambda i: (i, 0)),
        ],
        out_shape=[
            jax.ShapeDtypeStruct((N, D), jnp.float32),
            jax.ShapeDtypeStruct((N, 1), jnp.float32),
        ],
    )


def _make_agg_kernel(N, E, D, CH):
    EPW = E // NW
    NCHK = EPW // CH
    R = N // NS
    ZR = 25
    mesh = plsc.VectorSubcoreMesh(
        core_axis_name="c", subcore_axis_name="s", num_cores=NC, num_subcores=NS)

    @functools.partial(
        pl.kernel,
        out_type=jax.ShapeDtypeStruct((NC, N, D), jnp.float32),
        mesh=mesh,
        scratch_types=[
            pltpu.VMEM((NCHK, CH), jnp.int32),
            pltpu.VMEM((NCHK, CH), jnp.int32),
            pltpu.VMEM((CH, D), jnp.float32),
            pltpu.VMEM((CH, D), jnp.float32),
            pltpu.VMEM((25, D), jnp.float32),
            pltpu.VMEM_SHARED((N, D), jnp.float32),
            pltpu.SemaphoreType.DMA,
            pltpu.SemaphoreType.DMA,
            pltpu.SemaphoreType.DMA,
            pltpu.SemaphoreType.DMA,
            pltpu.SemaphoreType.DMA,
        ],
        compiler_params=pltpu.CompilerParams(
            needs_layout_passes=False, use_tc_tiling_on_sc=False),
    )
    def agg_kernel(y_hbm, src_hbm, dst_hbm, out_hbm,
                   src_v, dst_v, rows0_v, rows1_v, zbuf_v, acc_sh,
                   s0a, s0b, s1a, s1b, si):
        c = lax.axis_index("c")
        s = lax.axis_index("s")
        wid = s * NC + c

        pltpu.async_copy(src_hbm.at[pl.ds(wid * NCHK, NCHK)], src_v, si)
        pltpu.async_copy(dst_hbm.at[pl.ds(wid * NCHK, NCHK)], dst_v, si)

        zeros = jnp.zeros((LANES,), jnp.float32)

        def zrow(i, carry):
            def zcol(j, cc):
                zbuf_v[i, pl.ds(j * LANES, LANES)] = zeros
                return cc
            return lax.fori_loop(0, D // LANES, zcol, carry)

        lax.fori_loop(0, ZR, zrow, None)
        for k in range(R // ZR):
            pltpu.sync_copy(zbuf_v, acc_sh.at[pl.ds(s * R + k * ZR, ZR)])

        pltpu.make_async_copy(
            src_hbm.at[pl.ds(wid * NCHK, NCHK)], src_v, si).wait()
        pltpu.make_async_copy(
            dst_hbm.at[pl.ds(wid * NCHK, NCHK)], dst_v, si).wait()

        plsc.subcore_barrier()

        H = CH // 2

        def gather(j, buf, sa, sb):
            pltpu.async_copy(
                y_hbm.at[src_v.at[j, pl.ds(0, H)]], buf.at[pl.ds(0, H)], sa)
            pltpu.async_copy(
                y_hbm.at[src_v.at[j, pl.ds(H, H)]], buf.at[pl.ds(H, H)], sb)

        def drain(buf, sa, sb):
            pltpu.make_async_copy(
                y_hbm.at[pl.ds(0, H)], buf.at[pl.ds(0, H)], sa).wait()
            pltpu.make_async_copy(
                y_hbm.at[pl.ds(0, H)], buf.at[pl.ds(H, H)], sb).wait()

        def scat(j, buf):
            pltpu.sync_copy(buf, acc_sh.at[dst_v.at[j]], add=True)

        gather(0, rows0_v, s0a, s0b)

        def pair_body(g, carry):
            gather(2 * g + 1, rows1_v, s1a, s1b)
            drain(rows0_v, s0a, s0b)
            scat(2 * g, rows0_v)
            gather(jnp.minimum(2 * g + 2, NCHK - 1), rows0_v, s0a, s0b)
            drain(rows1_v, s1a, s1b)
            scat(2 * g + 1, rows1_v)
            return carry

        lax.fori_loop(0, NCHK // 2, pair_body, None)
        drain(rows0_v, s0a, s0b)
        if NCHK % 2 == 1:
            scat(NCHK - 1, rows0_v)

        plsc.subcore_barrier()

        pltpu.sync_copy(acc_sh.at[pl.ds(s * R, R)], out_hbm.at[c, pl.ds(s * R, R)])

    return agg_kernel


def _make_final_kernel(N, D, BLK):
    def final_body(x_ref, acc_ref, r_ref, wl_ref, wh_ref, wm_ref,
                   al_ref, ah_ref, am_ref, att_ref, out_ref):
        r = r_ref[...]
        low = (acc_ref[0] + acc_ref[1]) * r
        x = x_ref[...]
        high = x - low
        lowW = jnp.maximum(
            jnp.dot(low, wl_ref[...], preferred_element_type=jnp.float32), 0.0)
        highW = jnp.maximum(
            jnp.dot(high, wh_ref[...], preferred_element_type=jnp.float32), 0.0)
        mlpW = jnp.maximum(
            jnp.dot(x, wm_ref[...], preferred_element_type=jnp.float32), 0.0)

        l0 = jnp.dot(lowW, al_ref[...], preferred_element_type=jnp.float32)
        l1 = jnp.dot(highW, ah_ref[...], preferred_element_type=jnp.float32)
        l2 = jnp.dot(mlpW, am_ref[...], preferred_element_type=jnp.float32)

        s0 = 1.0 / (1.0 + jnp.exp(-l0))
        s1 = 1.0 / (1.0 + jnp.exp(-l1))
        s2 = 1.0 / (1.0 + jnp.exp(-l2))

        T_inv = 1.0 / 3.0
        m0 = (s0 * att_ref[0, 0] + s1 * att_ref[1, 0] + s2 * att_ref[2, 0]) * T_inv
        m1 = (s0 * att_ref[0, 1] + s1 * att_ref[1, 1] + s2 * att_ref[2, 1]) * T_inv
        m2 = (s0 * att_ref[0, 2] + s1 * att_ref[1, 2] + s2 * att_ref[2, 2]) * T_inv

        mx = jnp.maximum(jnp.maximum(m0, m1), m2)
        e0 = jnp.exp(m0 - mx)
        e1 = jnp.exp(m1 - mx)
        e2 = jnp.exp(m2 - mx)
        den = e0 + e1 + e2

        out_ref[...] = 3.0 * ((e0 / den) * lowW + (e1 / den) * highW +
                              (e2 / den) * mlpW)

    return pl.pallas_call(
        final_body,
        grid=(N // BLK,),
        in_specs=[
            pl.BlockSpec((BLK, D), lambda i: (i, 0)),
            pl.BlockSpec((NC, BLK, D), lambda i: (0, i, 0)),
            pl.BlockSpec((BLK, 1), lambda i: (i, 0)),
            pl.BlockSpec((D, D), lambda i: (0, 0)),
            pl.BlockSpec((D, D), lambda i: (0, 0)),
            pl.BlockSpec((D, D), lambda i: (0, 0)),
            pl.BlockSpec((D, 1), lambda i: (0, 0)),
            pl.BlockSpec((D, 1), lambda i: (0, 0)),
            pl.BlockSpec((D, 1), lambda i: (0, 0)),
            pl.BlockSpec(memory_space=pltpu.SMEM),
        ],
        out_specs=pl.BlockSpec((BLK, D), lambda i: (i, 0)),
        out_shape=jax.ShapeDtypeStruct((N, D), jnp.float32),
    )


def kernel(x, edge_index, W_low, W_high, W_mlp, a_low, a_high, a_mlp, att_vec):
    N, D = x.shape
    E = edge_index.shape[1]
    CH = 80
    BLK = 2000

    src = edge_index[0]
    dst = edge_index[1]
    src_ch = src.reshape(E // CH, CH)
    dst_ch = dst.reshape(E // CH, CH)

    deg_p = _make_deg_kernel(N, E)(src).reshape(NW, N)
    deg_pt = deg_p.T

    y, r = _make_prep_kernel(N, D, BLK)(x, deg_pt)
    acc = _make_agg_kernel(N, E, D, CH)(y, src_ch, dst_ch)

    return _make_final_kernel(N, D, BLK)(
        x, acc, r, W_low, W_high, W_mlp, a_low, a_high, a_mlp, att_vec)
